# baseline (device time: 123061 ns/iter reference)
import jax
import jax.numpy as jnp
from jax import lax
from jax.experimental import pallas as pl
from jax.experimental.pallas import tpu as pltpu

N_DEV = 16


def _gelu_f32(y):
    c = 0.7978845608028654
    return 0.5 * y * (1.0 + jnp.tanh(c * (y + 0.044715 * y * y * y)))


def kernel(x, w_mat):
    m_per, k = x.shape
    _, n_per = w_mat.shape
    x = x.astype(jnp.bfloat16)
    w = w_mat.astype(jnp.bfloat16)

    def body(x_ref, w_ref, out_ref, comm_ref, send_sems, recv_sems):
        my = lax.axis_index("i")
        left = (my - 1) % N_DEV
        right = (my + 1) % N_DEV

        barrier_sem = pltpu.get_barrier_semaphore()
        for nbr in (left, right):
            pl.semaphore_signal(
                barrier_sem, inc=1,
                device_id=(nbr,), device_id_type=pl.DeviceIdType.MESH,
            )
        pl.semaphore_wait(barrier_sem, 2)

        comm_ref[0, :, :] = x_ref[:, :]
        y0 = jnp.dot(x_ref[:, :], w_ref[:, :],
                     preferred_element_type=jnp.float32)
        out_ref[pl.ds(my * m_per, m_per), :] = _gelu_f32(y0)

        for h in range(N_DEV - 1):
            rdma = pltpu.make_async_remote_copy(
                src_ref=comm_ref.at[h],
                dst_ref=comm_ref.at[h + 1],
                send_sem=send_sems.at[h],
                recv_sem=recv_sems.at[h],
                device_id=(right,),
                device_id_type=pl.DeviceIdType.MESH,
            )
            rdma.start()
            rdma.wait()

            origin = (my - h - 1) % N_DEV
            yh = jnp.dot(comm_ref[h + 1, :, :], w_ref[:, :],
                         preferred_element_type=jnp.float32)
            out_ref[pl.ds(origin * m_per, m_per), :] = _gelu_f32(yh)

    return pl.pallas_call(
        body,
        out_shape=jax.ShapeDtypeStruct((N_DEV * m_per, n_per), jnp.float32),
        in_specs=[
            pl.BlockSpec(memory_space=pltpu.VMEM),
            pl.BlockSpec(memory_space=pltpu.VMEM),
        ],
        out_specs=pl.BlockSpec(memory_space=pltpu.VMEM),
        scratch_shapes=[
            pltpu.VMEM((N_DEV, m_per, k), jnp.bfloat16),
            pltpu.SemaphoreType.DMA((N_DEV - 1,)),
            pltpu.SemaphoreType.DMA((N_DEV - 1,)),
        ],
        compiler_params=pltpu.CompilerParams(collective_id=0),
    )(x, w)


# device time: 65713 ns/iter; 1.8727x vs baseline; 1.8727x over previous
import jax
import jax.numpy as jnp
from jax import lax
from jax.experimental import pallas as pl
from jax.experimental.pallas import tpu as pltpu

N_DEV = 16
R_HOPS = 8
L_HOPS = 7


def _gelu_f32(y):
    c = 0.7978845608028654
    return 0.5 * y * (1.0 + jnp.tanh(c * (y + 0.044715 * y * y * y)))


def kernel(x, w_mat):
    m_per, k = x.shape
    _, n_per = w_mat.shape
    x = x.astype(jnp.bfloat16)
    w = w_mat.astype(jnp.bfloat16)

    def _r_slot(r):
        return 0 if r == 0 else r
    def _l_slot(l):
        return 0 if l == 0 else 8 + l

    def body(x_ref, w_ref, out_ref, comm_ref, send_r, recv_r, send_l, recv_l):
        my = lax.axis_index("i")
        left = (my - 1) % N_DEV
        right = (my + 1) % N_DEV

        barrier_sem = pltpu.get_barrier_semaphore()
        for nbr in (left, right):
            pl.semaphore_signal(
                barrier_sem, inc=1,
                device_id=(nbr,), device_id_type=pl.DeviceIdType.MESH,
            )
        pl.semaphore_wait(barrier_sem, 2)

        comm_ref[0, :, :] = x_ref[:, :]

        def _send(chain_src_slot, chain_dst_slot, ssem, rsem, dst_dev):
            rdma = pltpu.make_async_remote_copy(
                src_ref=comm_ref.at[chain_src_slot],
                dst_ref=comm_ref.at[chain_dst_slot],
                send_sem=ssem,
                recv_sem=rsem,
                device_id=(dst_dev,),
                device_id_type=pl.DeviceIdType.MESH,
            )
            rdma.start()
            return rdma

        sends = []
        sends.append(_send(0, 1, send_r.at[0], recv_r.at[0], right))
        sends.append(_send(0, 9, send_l.at[0], recv_l.at[0], left))

        y0 = jnp.dot(x_ref[:, :], w_ref[:, :],
                     preferred_element_type=jnp.float32)
        out_ref[pl.ds(my * m_per, m_per), :] = _gelu_f32(y0)

        for s in range(R_HOPS):
            recv = pltpu.make_async_remote_copy(
                src_ref=comm_ref.at[s + 1], dst_ref=comm_ref.at[s + 1],
                send_sem=send_r.at[s], recv_sem=recv_r.at[s],
                device_id=(left,), device_id_type=pl.DeviceIdType.MESH,
            )
            recv.wait_recv()
            if s + 1 < R_HOPS:
                sends.append(_send(_r_slot(s + 1), s + 2,
                                   send_r.at[s + 1], recv_r.at[s + 1], right))

            have_l = s < L_HOPS
            if have_l:
                recv = pltpu.make_async_remote_copy(
                    src_ref=comm_ref.at[9 + s], dst_ref=comm_ref.at[9 + s],
                    send_sem=send_l.at[s], recv_sem=recv_l.at[s],
                    device_id=(right,), device_id_type=pl.DeviceIdType.MESH,
                )
                recv.wait_recv()
                if s + 1 < L_HOPS:
                    sends.append(_send(_l_slot(s + 1), 9 + s + 1,
                                       send_l.at[s + 1], recv_l.at[s + 1],
                                       left))

            origin_r = (my - s - 1) % N_DEV
            yr = jnp.dot(comm_ref[s + 1, :, :], w_ref[:, :],
                         preferred_element_type=jnp.float32)
            out_ref[pl.ds(origin_r * m_per, m_per), :] = _gelu_f32(yr)
            if have_l:
                origin_l = (my + s + 1) % N_DEV
                yl = jnp.dot(comm_ref[9 + s, :, :], w_ref[:, :],
                             preferred_element_type=jnp.float32)
                out_ref[pl.ds(origin_l * m_per, m_per), :] = _gelu_f32(yl)

        for rdma in sends:
            rdma.wait_send()

    return pl.pallas_call(
        body,
        out_shape=jax.ShapeDtypeStruct((N_DEV * m_per, n_per), jnp.float32),
        in_specs=[
            pl.BlockSpec(memory_space=pltpu.VMEM),
            pl.BlockSpec(memory_space=pltpu.VMEM),
        ],
        out_specs=pl.BlockSpec(memory_space=pltpu.VMEM),
        scratch_shapes=[
            pltpu.VMEM((N_DEV, m_per, k), jnp.bfloat16),
            pltpu.SemaphoreType.DMA((R_HOPS,)),
            pltpu.SemaphoreType.DMA((R_HOPS,)),
            pltpu.SemaphoreType.DMA((L_HOPS,)),
            pltpu.SemaphoreType.DMA((L_HOPS,)),
        ],
        compiler_params=pltpu.CompilerParams(collective_id=0),
    )(x, w)


# device time: 56220 ns/iter; 2.1889x vs baseline; 1.1689x over previous
import jax
import jax.numpy as jnp
from jax import lax
from jax.experimental import pallas as pl
from jax.experimental.pallas import tpu as pltpu

N_DEV = 16
R_HOPS = 8
L_HOPS = 7
SUB = 2


def _gelu_f32(y):
    c = 0.7978845608028654
    return 0.5 * y * (1.0 + jnp.tanh(c * (y + 0.044715 * y * y * y)))


def kernel(x, w_mat):
    m_per, k = x.shape
    _, n_per = w_mat.shape
    msub = m_per // SUB
    x = x.astype(jnp.bfloat16)
    w = w_mat.astype(jnp.bfloat16)

    def _r_src(s):
        return 0 if s == 0 else s
    def _l_src(s):
        return 0 if s == 0 else 8 + s

    def body(x_ref, w_ref, out_ref, comm_ref, send_r, recv_r, send_l, recv_l):
        my = lax.axis_index("i")
        left = (my - 1) % N_DEV
        right = (my + 1) % N_DEV

        barrier_sem = pltpu.get_barrier_semaphore()
        for nbr in (left, right):
            pl.semaphore_signal(
                barrier_sem, inc=1,
                device_id=(nbr,), device_id_type=pl.DeviceIdType.MESH,
            )
        pl.semaphore_wait(barrier_sem, 2)

        comm_ref[0, :, :] = x_ref[:, :]

        sends = []

        def _send(src_slot, dst_slot, j, ssem, rsem, dst_dev):
            rows = pl.ds(j * msub, msub)
            rdma = pltpu.make_async_remote_copy(
                src_ref=comm_ref.at[src_slot, rows],
                dst_ref=comm_ref.at[dst_slot, rows],
                send_sem=ssem, recv_sem=rsem,
                device_id=(dst_dev,),
                device_id_type=pl.DeviceIdType.MESH,
            )
            rdma.start()
            sends.append(rdma)

        def _wait_recv(dst_slot, j, ssem, rsem):
            rows = pl.ds(j * msub, msub)
            rdma = pltpu.make_async_remote_copy(
                src_ref=comm_ref.at[dst_slot, rows],
                dst_ref=comm_ref.at[dst_slot, rows],
                send_sem=ssem, recv_sem=rsem,
                device_id=(left,), device_id_type=pl.DeviceIdType.MESH,
            )
            rdma.wait_recv()

        for j in range(SUB):
            _send(0, 1, j, send_r.at[0, j], recv_r.at[0, j], right)
            _send(0, 9, j, send_l.at[0, j], recv_l.at[0, j], left)

        y0 = jnp.dot(x_ref[:, :], w_ref[:, :],
                     preferred_element_type=jnp.float32)
        out_ref[pl.ds(my * m_per, m_per), :] = _gelu_f32(y0)

        for s in range(R_HOPS):
            have_l = s < L_HOPS
            for j in range(SUB):
                _wait_recv(s + 1, j, send_r.at[s, j], recv_r.at[s, j])
                if s + 1 < R_HOPS:
                    _send(s + 1, s + 2, j,
                          send_r.at[s + 1, j], recv_r.at[s + 1, j], right)
                if have_l:
                    _wait_recv(9 + s, j, send_l.at[s, j], recv_l.at[s, j])
                    if s + 1 < L_HOPS:
                        _send(9 + s, 9 + s + 1, j,
                              send_l.at[s + 1, j], recv_l.at[s + 1, j], left)

            origin_r = (my - s - 1) % N_DEV
            yr = jnp.dot(comm_ref[s + 1, :, :], w_ref[:, :],
                         preferred_element_type=jnp.float32)
            out_ref[pl.ds(origin_r * m_per, m_per), :] = _gelu_f32(yr)
            if have_l:
                origin_l = (my + s + 1) % N_DEV
                yl = jnp.dot(comm_ref[9 + s, :, :], w_ref[:, :],
                             preferred_element_type=jnp.float32)
                out_ref[pl.ds(origin_l * m_per, m_per), :] = _gelu_f32(yl)

        for rdma in sends:
            rdma.wait_send()

    return pl.pallas_call(
        body,
        out_shape=jax.ShapeDtypeStruct((N_DEV * m_per, n_per), jnp.float32),
        in_specs=[
            pl.BlockSpec(memory_space=pltpu.VMEM),
            pl.BlockSpec(memory_space=pltpu.VMEM),
        ],
        out_specs=pl.BlockSpec(memory_space=pltpu.VMEM),
        scratch_shapes=[
            pltpu.VMEM((N_DEV, m_per, k), jnp.bfloat16),
            pltpu.SemaphoreType.DMA((R_HOPS, SUB)),
            pltpu.SemaphoreType.DMA((R_HOPS, SUB)),
            pltpu.SemaphoreType.DMA((L_HOPS, SUB)),
            pltpu.SemaphoreType.DMA((L_HOPS, SUB)),
        ],
        compiler_params=pltpu.CompilerParams(collective_id=0),
    )(x, w)


# device time: 55466 ns/iter; 2.2187x vs baseline; 1.0136x over previous
import jax
import jax.numpy as jnp
from jax import lax
from jax.experimental import pallas as pl
from jax.experimental.pallas import tpu as pltpu

N_DEV = 16
HOPS = 8
SUB = 2


def _gelu_f32(y):
    c = 0.7978845608028654
    return 0.5 * y * (1.0 + jnp.tanh(c * (y + 0.044715 * y * y * y)))


def kernel(x, w_mat):
    m_per, k = x.shape
    _, n_per = w_mat.shape
    msub = m_per // SUB

    def _r_active(s, j):
        return s < HOPS - 1 or j == 0

    def _l_active(s, j):
        return s < HOPS - 1 or j == 1

    def _r_dst(s):
        return s + 1

    def _l_dst(s):
        return 8 if s == HOPS - 1 else 9 + s

    def _r_src(s):
        return 0 if s == 0 else s

    def _l_src(s):
        return 0 if s == 0 else 8 + s

    def body(x_ref, w_ref, out_ref, comm_ref, w_bf, send_r, recv_r,
             send_l, recv_l):
        my = lax.axis_index("i")
        left = (my - 1) % N_DEV
        right = (my + 1) % N_DEV

        barrier_sem = pltpu.get_barrier_semaphore()
        for nbr in (left, right):
            pl.semaphore_signal(
                barrier_sem, inc=1,
                device_id=(nbr,), device_id_type=pl.DeviceIdType.MESH,
            )
        pl.semaphore_wait(barrier_sem, 2)

        comm_ref[0, :, :] = x_ref[:, :].astype(jnp.bfloat16)

        sends = []

        def _send(src_slot, dst_slot, j, ssem, rsem, dst_dev):
            rows = pl.ds(j * msub, msub)
            rdma = pltpu.make_async_remote_copy(
                src_ref=comm_ref.at[src_slot, rows],
                dst_ref=comm_ref.at[dst_slot, rows],
                send_sem=ssem, recv_sem=rsem,
                device_id=(dst_dev,),
                device_id_type=pl.DeviceIdType.MESH,
            )
            rdma.start()
            sends.append(rdma)

        def _wait_recv(dst_slot, j, ssem, rsem):
            rows = pl.ds(j * msub, msub)
            rdma = pltpu.make_async_remote_copy(
                src_ref=comm_ref.at[dst_slot, rows],
                dst_ref=comm_ref.at[dst_slot, rows],
                send_sem=ssem, recv_sem=rsem,
                device_id=(left,), device_id_type=pl.DeviceIdType.MESH,
            )
            rdma.wait_recv()

        for j in range(SUB):
            _send(0, _r_dst(0), j, send_r.at[0, j], recv_r.at[0, j], right)
            _send(0, _l_dst(0), j, send_l.at[0, j], recv_l.at[0, j], left)

        w_bf[:, :] = w_ref[:, :].astype(jnp.bfloat16)
        y0 = jnp.dot(comm_ref[0, :, :], w_bf[:, :],
                     preferred_element_type=jnp.float32)
        out_ref[pl.ds(my * m_per, m_per), :] = _gelu_f32(y0)

        for s in range(HOPS):
            for j in range(SUB):
                if _r_active(s, j):
                    _wait_recv(_r_dst(s), j, send_r.at[s, j], recv_r.at[s, j])
                    if s + 1 < HOPS and _r_active(s + 1, j):
                        _send(_r_src(s + 1), _r_dst(s + 1), j,
                              send_r.at[s + 1, j], recv_r.at[s + 1, j], right)
                if _l_active(s, j):
                    _wait_recv(_l_dst(s), j, send_l.at[s, j], recv_l.at[s, j])
                    if s + 1 < HOPS and _l_active(s + 1, j):
                        _send(_l_src(s + 1), _l_dst(s + 1), j,
                              send_l.at[s + 1, j], recv_l.at[s + 1, j], left)

            if s < HOPS - 1:
                origin_r = (my - s - 1) % N_DEV
                yr = jnp.dot(comm_ref[s + 1, :, :], w_bf[:, :],
                             preferred_element_type=jnp.float32)
                out_ref[pl.ds(origin_r * m_per, m_per), :] = _gelu_f32(yr)
                origin_l = (my + s + 1) % N_DEV
                yl = jnp.dot(comm_ref[9 + s, :, :], w_bf[:, :],
                             preferred_element_type=jnp.float32)
                out_ref[pl.ds(origin_l * m_per, m_per), :] = _gelu_f32(yl)
            else:
                origin_a = (my + HOPS) % N_DEV
                ya = jnp.dot(comm_ref[8, :, :], w_bf[:, :],
                             preferred_element_type=jnp.float32)
                out_ref[pl.ds(origin_a * m_per, m_per), :] = _gelu_f32(ya)

        for rdma in sends:
            rdma.wait_send()

    return pl.pallas_call(
        body,
        out_shape=jax.ShapeDtypeStruct((N_DEV * m_per, n_per), jnp.float32),
        in_specs=[
            pl.BlockSpec(memory_space=pltpu.VMEM),
            pl.BlockSpec(memory_space=pltpu.VMEM),
        ],
        out_specs=pl.BlockSpec(memory_space=pltpu.VMEM),
        scratch_shapes=[
            pltpu.VMEM((N_DEV, m_per, k), jnp.bfloat16),
            pltpu.VMEM((k, n_per), jnp.bfloat16),
            pltpu.SemaphoreType.DMA((HOPS, SUB)),
            pltpu.SemaphoreType.DMA((HOPS, SUB)),
            pltpu.SemaphoreType.DMA((HOPS, SUB)),
            pltpu.SemaphoreType.DMA((HOPS, SUB)),
        ],
        compiler_params=pltpu.CompilerParams(collective_id=0),
    )(x, w_mat)
